# baseline (speedup 1.0000x reference)
import functools

import jax
import jax.numpy as jnp
from jax import lax
from jax.experimental import pallas as pl
from jax.experimental.pallas import tpu as pltpu
from jax.experimental.pallas import tpu_sc as plsc

NC = 2
NS = 16
NW = NC * NS

IW = 80
ZR = 160


def _sc_segment_sum(edge_attr, dest3d, npad):
    E, D = edge_attr.shape
    e_pt = E // NW
    rw = e_pt // IW
    npair = rw // 2
    tail = rw - 2 * npair
    rt = npad // NS
    ro = (rw // 2) // 8 * 8
    ic = ((rw - ro + 7) // 8) * 8

    mesh = plsc.VectorSubcoreMesh(
        core_axis_name="c", subcore_axis_name="s", num_cores=NC, num_subcores=NS
    )

    @functools.partial(
        pl.kernel,
        out_type=jax.ShapeDtypeStruct((NC, npad, D), jnp.float32),
        mesh=mesh,
        scratch_types=[
            pltpu.VMEM((2 * IW, D), jnp.float32),
            pltpu.VMEM((ic, IW), jnp.int32),
            pltpu.VMEM_SHARED((npad, D), jnp.float32),
            pltpu.SemaphoreType.DMA,
            pltpu.SemaphoreType.DMA,
            pltpu.SemaphoreType.DMA,
            pltpu.SemaphoreType.DMA,
        ],
    )
    def sc_agg(ea_hbm, di_hbm, out_hbm, rows_v, idx_v, acc_sh,
               sem_a, sem_b, sem_s, sem_t):
        c = lax.axis_index("c")
        s = lax.axis_index("s")
        wid = c * NS + s
        rows_a = rows_v.at[pl.ds(0, IW)]
        rows_b = rows_v.at[pl.ds(IW, IW)]

        pltpu.sync_copy(di_hbm.at[wid].at[pl.ds(0, ic)], idx_v)

        zero16 = jnp.zeros((16,), jnp.float32)

        def zbody(i, carry):
            for j in range(D // 16):
                rows_v[i, pl.ds(j * 16, 16)] = zero16
            return carry

        lax.fori_loop(0, ZR, zbody, 0)
        nfull = rt // ZR
        for r in range(nfull):
            pltpu.sync_copy(
                rows_v.at[pl.ds(0, ZR)],
                acc_sh.at[pl.ds(s * rt + r * ZR, ZR)],
            )
        if rt % ZR:
            pltpu.sync_copy(
                rows_v.at[pl.ds(0, rt % ZR)],
                acc_sh.at[pl.ds(s * rt + nfull * ZR, rt % ZR)],
            )
        plsc.subcore_barrier()

        def load(k, buf, sem):
            pltpu.async_copy(
                ea_hbm.at[pl.ds(wid * e_pt + k * IW, IW)], buf, sem
            )

        def wait_load(buf, sem):
            pltpu.make_async_copy(
                ea_hbm.at[pl.ds(0, IW)], buf, sem
            ).wait()

        def fire_scatter(buf, k, sem):
            irow = jnp.where(k < ro, k, k - ro)
            pltpu.async_copy(
                buf,
                acc_sh.at[idx_v.at[irow]],
                sem,
                add=True,
            )

        def drain_scatter(buf, sem):
            pltpu.make_async_copy(buf, acc_sh.at[pl.ds(0, IW)], sem).wait()

        load(0, rows_a, sem_a)

        def pair(kk, carry):
            k0 = 2 * kk

            @pl.when(k0 == ro)
            def _():
                pltpu.sync_copy(
                    di_hbm.at[wid].at[pl.ds(ro, ic)],
                    idx_v,
                )

            load(k0 + 1, rows_b, sem_b)
            wait_load(rows_a, sem_a)
            fire_scatter(rows_a, k0, sem_s)
            wait_load(rows_b, sem_b)
            fire_scatter(rows_b, k0 + 1, sem_t)
            drain_scatter(rows_a, sem_s)

            @pl.when(k0 + 2 < rw)
            def _():
                load(k0 + 2, rows_a, sem_a)

            drain_scatter(rows_b, sem_t)
            return carry

        lax.fori_loop(0, npair, pair, 0)
        if tail:
            wait_load(rows_a, sem_a)
            fire_scatter(rows_a, rw - 1, sem_s)
            drain_scatter(rows_a, sem_s)
        plsc.subcore_barrier()

        pltpu.sync_copy(
            acc_sh.at[pl.ds(s * rt, rt)],
            out_hbm.at[c].at[pl.ds(s * rt, rt)],
        )

    return sc_agg(edge_attr, dest3d)


def _u_body(x_ref, f_ref, wx_ref, wf_ref, b0_ref, u_ref):
    u = jnp.dot(x_ref[...], wx_ref[...], preferred_element_type=jnp.float32)
    u = u + jnp.dot(f_ref[...], wf_ref[...],
                    preferred_element_type=jnp.float32)
    u_ref[...] = u + b0_ref[...]


def _mlp_body(u_ref, p0_ref, p1_ref, wa_ref, w1_ref, b1_ref, w2_ref, b2_ref,
              o_ref):
    agg = p0_ref[0] + p1_ref[0]
    h = u_ref[...] + jnp.dot(agg, wa_ref[...],
                             preferred_element_type=jnp.float32)
    h = h * jax.nn.sigmoid(h)
    h = jnp.dot(h, w1_ref[...],
                preferred_element_type=jnp.float32) + b1_ref[...]
    h = h * jax.nn.sigmoid(h)
    o_ref[...] = jnp.dot(h, w2_ref[...],
                         preferred_element_type=jnp.float32) + b2_ref[...]


@jax.jit
def kernel(x, edge_index, edge_attr, f, W0, b0, W1, b1, W2, b2):
    N, D = x.shape
    E = edge_attr.shape[0]
    F = f.shape[1]
    H = W1.shape[0]

    algn = NS * 8
    npad = ((N + algn - 1) // algn) * algn

    rw = E // (NW * IW)
    rw_pad = ((rw + 7) // 8) * 8
    dest3d = edge_index[1].reshape(NW, rw, IW)
    if rw_pad != rw:
        dest3d = jnp.pad(dest3d, ((0, 0), (0, rw_pad - rw), (0, 0)))
    partials = _sc_segment_sum(edge_attr, dest3d, npad)

    R = 2000
    nb = N // R
    Wx = W0[:D]
    Wa = W0[D:D + H]
    Wf = W0[D + H:]
    b0r = b0.reshape(1, H)
    b1r = b1.reshape(1, H)
    b2r = b2.reshape(1, H)

    const = lambda i: (0, 0)
    row = lambda i: (i, 0)
    u = pl.pallas_call(
        _u_body,
        grid=(nb,),
        in_specs=[
            pl.BlockSpec((R, D), row),
            pl.BlockSpec((R, F), row),
            pl.BlockSpec((D, H), const),
            pl.BlockSpec((F, H), const),
            pl.BlockSpec((1, H), const),
        ],
        out_specs=pl.BlockSpec((R, H), row),
        out_shape=jax.ShapeDtypeStruct((N, H), jnp.float32),
    )(x, f, Wx, Wf, b0r)

    out = pl.pallas_call(
        _mlp_body,
        grid=(nb,),
        in_specs=[
            pl.BlockSpec((R, H), row),
            pl.BlockSpec((1, R, D), lambda i: (0, i, 0)),
            pl.BlockSpec((1, R, D), lambda i: (1, i, 0)),
            pl.BlockSpec((H, H), const),
            pl.BlockSpec((H, H), const),
            pl.BlockSpec((1, H), const),
            pl.BlockSpec((H, H), const),
            pl.BlockSpec((1, H), const),
        ],
        out_specs=pl.BlockSpec((R, H), row),
        out_shape=jax.ShapeDtypeStruct((N, H), jnp.float32),
    )(u, partials, partials, Wa, W1, b1r, W2, b2r)
    return out


# speedup vs baseline: 1.2302x; 1.2302x over previous
import functools

import jax
import jax.numpy as jnp
from jax import lax
from jax.experimental import pallas as pl
from jax.experimental.pallas import tpu as pltpu
from jax.experimental.pallas import tpu_sc as plsc

NC = 2
NS = 16
NW = NC * NS

IW = 80
ZR = 160


def _sc_segment_sum(edge_attr, dest3d, npad):
    E, D = edge_attr.shape
    e_pt = E // NW
    rw = e_pt // IW
    npair = rw // 2
    tail = rw - 2 * npair
    rt = npad // NS
    ro = (rw // 2) // 8 * 8
    ic = ((rw - ro + 7) // 8) * 8

    mesh = plsc.VectorSubcoreMesh(
        core_axis_name="c", subcore_axis_name="s", num_cores=NC, num_subcores=NS
    )

    @functools.partial(
        pl.kernel,
        out_type=jax.ShapeDtypeStruct((NC, npad, D), jnp.float32),
        mesh=mesh,
        scratch_types=[
            pltpu.VMEM((2 * IW, D), jnp.float32),
            pltpu.VMEM((ic, IW), jnp.int32),
            pltpu.VMEM_SHARED((npad, D), jnp.float32),
            pltpu.SemaphoreType.DMA,
            pltpu.SemaphoreType.DMA,
        ],
    )
    def sc_agg(ea_hbm, di_hbm, out_hbm, rows_v, idx_v, acc_sh,
               sem_a, sem_b):
        c = lax.axis_index("c")
        s = lax.axis_index("s")
        wid = c * NS + s
        rows_a = rows_v.at[pl.ds(0, IW)]
        rows_b = rows_v.at[pl.ds(IW, IW)]

        pltpu.sync_copy(di_hbm.at[wid].at[pl.ds(0, ic)], idx_v)

        zero16 = jnp.zeros((16,), jnp.float32)

        def zbody(i, carry):
            for j in range(D // 16):
                rows_v[i, pl.ds(j * 16, 16)] = zero16
            return carry

        lax.fori_loop(0, ZR, zbody, 0)
        nfull = rt // ZR
        for r in range(nfull):
            pltpu.sync_copy(
                rows_v.at[pl.ds(0, ZR)],
                acc_sh.at[pl.ds(s * rt + r * ZR, ZR)],
            )
        if rt % ZR:
            pltpu.sync_copy(
                rows_v.at[pl.ds(0, rt % ZR)],
                acc_sh.at[pl.ds(s * rt + nfull * ZR, rt % ZR)],
            )
        plsc.subcore_barrier()

        def load(k, buf, sem):
            pltpu.async_copy(
                ea_hbm.at[pl.ds(wid * e_pt + k * IW, IW)], buf, sem
            )

        def wait_load(buf, sem):
            pltpu.make_async_copy(
                ea_hbm.at[pl.ds(0, IW)], buf, sem
            ).wait()

        def scatter(buf, k):
            irow = jnp.where(k < ro, k, k - ro)
            pltpu.sync_copy(
                buf,
                acc_sh.at[idx_v.at[irow]],
                add=True,
            )

        load(0, rows_a, sem_a)

        def pair(kk, carry):
            k0 = 2 * kk

            @pl.when(k0 == ro)
            def _():
                pltpu.sync_copy(
                    di_hbm.at[wid].at[pl.ds(ro, ic)],
                    idx_v,
                )

            load(k0 + 1, rows_b, sem_b)
            wait_load(rows_a, sem_a)
            scatter(rows_a, k0)

            @pl.when(k0 + 2 < rw)
            def _():
                load(k0 + 2, rows_a, sem_a)

            wait_load(rows_b, sem_b)
            scatter(rows_b, k0 + 1)
            return carry

        lax.fori_loop(0, npair, pair, 0)
        if tail:
            wait_load(rows_a, sem_a)
            scatter(rows_a, rw - 1)
        plsc.subcore_barrier()

        pltpu.sync_copy(
            acc_sh.at[pl.ds(s * rt, rt)],
            out_hbm.at[c].at[pl.ds(s * rt, rt)],
        )

    return sc_agg(edge_attr, dest3d)


def _u_body(x_ref, f_ref, wx_ref, wf_ref, b0_ref, u_ref):
    u = jnp.dot(x_ref[...], wx_ref[...], preferred_element_type=jnp.float32)
    u = u + jnp.dot(f_ref[...], wf_ref[...],
                    preferred_element_type=jnp.float32)
    u_ref[...] = u + b0_ref[...]


def _mlp_body(u_ref, p0_ref, p1_ref, wa_ref, w1_ref, b1_ref, w2_ref, b2_ref,
              o_ref):
    agg = p0_ref[0] + p1_ref[0]
    h = u_ref[...] + jnp.dot(agg, wa_ref[...],
                             preferred_element_type=jnp.float32)
    h = h * jax.nn.sigmoid(h)
    h = jnp.dot(h, w1_ref[...],
                preferred_element_type=jnp.float32) + b1_ref[...]
    h = h * jax.nn.sigmoid(h)
    o_ref[...] = jnp.dot(h, w2_ref[...],
                         preferred_element_type=jnp.float32) + b2_ref[...]


@jax.jit
def kernel(x, edge_index, edge_attr, f, W0, b0, W1, b1, W2, b2):
    N, D = x.shape
    E = edge_attr.shape[0]
    F = f.shape[1]
    H = W1.shape[0]

    algn = NS * 8
    npad = ((N + algn - 1) // algn) * algn

    rw = E // (NW * IW)
    rw_pad = ((rw + 7) // 8) * 8
    dest3d = edge_index[1].reshape(NW, rw, IW)
    if rw_pad != rw:
        dest3d = jnp.pad(dest3d, ((0, 0), (0, rw_pad - rw), (0, 0)))
    partials = _sc_segment_sum(edge_attr, dest3d, npad)

    R = 2000
    nb = N // R
    Wx = W0[:D]
    Wa = W0[D:D + H]
    Wf = W0[D + H:]
    b0r = b0.reshape(1, H)
    b1r = b1.reshape(1, H)
    b2r = b2.reshape(1, H)

    const = lambda i: (0, 0)
    row = lambda i: (i, 0)
    u = pl.pallas_call(
        _u_body,
        grid=(nb,),
        in_specs=[
            pl.BlockSpec((R, D), row),
            pl.BlockSpec((R, F), row),
            pl.BlockSpec((D, H), const),
            pl.BlockSpec((F, H), const),
            pl.BlockSpec((1, H), const),
        ],
        out_specs=pl.BlockSpec((R, H), row),
        out_shape=jax.ShapeDtypeStruct((N, H), jnp.float32),
    )(x, f, Wx, Wf, b0r)

    out = pl.pallas_call(
        _mlp_body,
        grid=(nb,),
        in_specs=[
            pl.BlockSpec((R, H), row),
            pl.BlockSpec((1, R, D), lambda i: (0, i, 0)),
            pl.BlockSpec((1, R, D), lambda i: (1, i, 0)),
            pl.BlockSpec((H, H), const),
            pl.BlockSpec((H, H), const),
            pl.BlockSpec((1, H), const),
            pl.BlockSpec((H, H), const),
            pl.BlockSpec((1, H), const),
        ],
        out_specs=pl.BlockSpec((R, H), row),
        out_shape=jax.ShapeDtypeStruct((N, H), jnp.float32),
    )(u, partials, partials, Wa, W1, b1r, W2, b2r)
    return out


# speedup vs baseline: 1.2840x; 1.0437x over previous
import functools

import jax
import jax.numpy as jnp
from jax import lax
from jax.experimental import pallas as pl
from jax.experimental.pallas import tpu as pltpu
from jax.experimental.pallas import tpu_sc as plsc

NC = 2
NS = 16
NW = NC * NS

IW = 104
ZR = 160


def _sc_segment_sum(edge_attr, dest3d, npad):
    E, D = edge_attr.shape
    e_pt = E // NW
    nfc = e_pt // IW
    te = e_pt - nfc * IW
    rw = nfc + (1 if te else 0)
    npair = nfc // 2
    rt = npad // NS
    ro = (rw // 2) // 8 * 8
    ic = ((rw - ro + 7) // 8) * 8

    mesh = plsc.VectorSubcoreMesh(
        core_axis_name="c", subcore_axis_name="s", num_cores=NC, num_subcores=NS
    )

    @functools.partial(
        pl.kernel,
        out_type=jax.ShapeDtypeStruct((NC, npad, D), jnp.float32),
        mesh=mesh,
        scratch_types=[
            pltpu.VMEM((2 * IW, D), jnp.float32),
            pltpu.VMEM((ic, IW), jnp.int32),
            pltpu.VMEM_SHARED((npad, D), jnp.float32),
            pltpu.SemaphoreType.DMA,
            pltpu.SemaphoreType.DMA,
        ],
    )
    def sc_agg(ea_hbm, di_hbm, out_hbm, rows_v, idx_v, acc_sh,
               sem_a, sem_b):
        c = lax.axis_index("c")
        s = lax.axis_index("s")
        wid = c * NS + s
        rows_a = rows_v.at[pl.ds(0, IW)]
        rows_b = rows_v.at[pl.ds(IW, IW)]

        pltpu.sync_copy(di_hbm.at[wid].at[pl.ds(0, ic)], idx_v)

        zero16 = jnp.zeros((16,), jnp.float32)

        def zbody(i, carry):
            for j in range(D // 16):
                rows_v[i, pl.ds(j * 16, 16)] = zero16
            return carry

        lax.fori_loop(0, ZR, zbody, 0)
        nfull = rt // ZR
        for r in range(nfull):
            pltpu.sync_copy(
                rows_v.at[pl.ds(0, ZR)],
                acc_sh.at[pl.ds(s * rt + r * ZR, ZR)],
            )
        if rt % ZR:
            pltpu.sync_copy(
                rows_v.at[pl.ds(0, rt % ZR)],
                acc_sh.at[pl.ds(s * rt + nfull * ZR, rt % ZR)],
            )
        plsc.subcore_barrier()

        def load(k, buf, sem):
            pltpu.async_copy(
                ea_hbm.at[pl.ds(wid * e_pt + k * IW, IW)], buf, sem
            )

        def wait_load(buf, sem):
            pltpu.make_async_copy(
                ea_hbm.at[pl.ds(0, IW)], buf, sem
            ).wait()

        def scatter(buf, k):
            irow = jnp.where(k < ro, k, k - ro)
            pltpu.sync_copy(
                buf,
                acc_sh.at[idx_v.at[irow]],
                add=True,
            )

        load(0, rows_a, sem_a)

        def pair(kk, carry):
            k0 = 2 * kk

            @pl.when(k0 == ro)
            def _():
                pltpu.sync_copy(
                    di_hbm.at[wid].at[pl.ds(ro, ic)],
                    idx_v,
                )

            load(k0 + 1, rows_b, sem_b)
            wait_load(rows_a, sem_a)
            scatter(rows_a, k0)

            @pl.when(k0 + 2 < nfc)
            def _():
                load(k0 + 2, rows_a, sem_a)

            wait_load(rows_b, sem_b)
            scatter(rows_b, k0 + 1)
            return carry

        lax.fori_loop(0, npair, pair, 0)
        if te:
            pltpu.sync_copy(
                ea_hbm.at[pl.ds(wid * e_pt + nfc * IW, te)],
                rows_v.at[pl.ds(0, te)],
            )

            def tzero(i, carry):
                for j in range(D // 16):
                    rows_v[i, pl.ds(j * 16, 16)] = zero16
                return carry

            lax.fori_loop(te, IW, tzero, 0)
            scatter(rows_a, rw - 1)
        plsc.subcore_barrier()

        pltpu.sync_copy(
            acc_sh.at[pl.ds(s * rt, rt)],
            out_hbm.at[c].at[pl.ds(s * rt, rt)],
        )

    return sc_agg(edge_attr, dest3d)


def _u_body(x_ref, f_ref, wx_ref, wf_ref, b0_ref, u_ref):
    u = jnp.dot(x_ref[...], wx_ref[...], preferred_element_type=jnp.float32)
    u = u + jnp.dot(f_ref[...], wf_ref[...],
                    preferred_element_type=jnp.float32)
    u_ref[...] = u + b0_ref[...]


def _mlp_body(u_ref, p0_ref, p1_ref, wa_ref, w1_ref, b1_ref, w2_ref, b2_ref,
              o_ref):
    agg = p0_ref[0] + p1_ref[0]
    h = u_ref[...] + jnp.dot(agg, wa_ref[...],
                             preferred_element_type=jnp.float32)
    h = h * jax.nn.sigmoid(h)
    h = jnp.dot(h, w1_ref[...],
                preferred_element_type=jnp.float32) + b1_ref[...]
    h = h * jax.nn.sigmoid(h)
    o_ref[...] = jnp.dot(h, w2_ref[...],
                         preferred_element_type=jnp.float32) + b2_ref[...]


@jax.jit
def kernel(x, edge_index, edge_attr, f, W0, b0, W1, b1, W2, b2):
    N, D = x.shape
    E = edge_attr.shape[0]
    F = f.shape[1]
    H = W1.shape[0]

    algn = NS * 8
    npad = ((N + algn - 1) // algn) * algn

    e_pt = E // NW
    nfc = e_pt // IW
    te = e_pt - nfc * IW
    rw = nfc + (1 if te else 0)
    rw_pad = ((rw + 7) // 8) * 8
    dest2 = edge_index[1].reshape(NW, e_pt)
    if te:
        dest2 = jnp.pad(dest2, ((0, 0), (0, rw * IW - e_pt)))
    dest3d = dest2.reshape(NW, rw, IW)
    if rw_pad != rw:
        dest3d = jnp.pad(dest3d, ((0, 0), (0, rw_pad - rw), (0, 0)))
    partials = _sc_segment_sum(edge_attr, dest3d, npad)

    R = 2000
    nb = N // R
    Wx = W0[:D]
    Wa = W0[D:D + H]
    Wf = W0[D + H:]
    b0r = b0.reshape(1, H)
    b1r = b1.reshape(1, H)
    b2r = b2.reshape(1, H)

    const = lambda i: (0, 0)
    row = lambda i: (i, 0)
    u = pl.pallas_call(
        _u_body,
        grid=(nb,),
        in_specs=[
            pl.BlockSpec((R, D), row),
            pl.BlockSpec((R, F), row),
            pl.BlockSpec((D, H), const),
            pl.BlockSpec((F, H), const),
            pl.BlockSpec((1, H), const),
        ],
        out_specs=pl.BlockSpec((R, H), row),
        out_shape=jax.ShapeDtypeStruct((N, H), jnp.float32),
    )(x, f, Wx, Wf, b0r)

    out = pl.pallas_call(
        _mlp_body,
        grid=(nb,),
        in_specs=[
            pl.BlockSpec((R, H), row),
            pl.BlockSpec((1, R, D), lambda i: (0, i, 0)),
            pl.BlockSpec((1, R, D), lambda i: (1, i, 0)),
            pl.BlockSpec((H, H), const),
            pl.BlockSpec((H, H), const),
            pl.BlockSpec((1, H), const),
            pl.BlockSpec((H, H), const),
            pl.BlockSpec((1, H), const),
        ],
        out_specs=pl.BlockSpec((R, H), row),
        out_shape=jax.ShapeDtypeStruct((N, H), jnp.float32),
    )(u, partials, partials, Wa, W1, b1r, W2, b2r)
    return out


# speedup vs baseline: 1.2977x; 1.0107x over previous
import functools

import jax
import jax.numpy as jnp
from jax import lax
from jax.experimental import pallas as pl
from jax.experimental.pallas import tpu as pltpu
from jax.experimental.pallas import tpu_sc as plsc

NC = 2
NS = 16
NW = NC * NS

IW = 112
ZR = 160


def _sc_segment_sum(edge_attr, dest3d, npad):
    E, D = edge_attr.shape
    e_pt = E // NW
    nfc = e_pt // IW
    te = e_pt - nfc * IW
    rw = nfc + (1 if te else 0)
    npair = nfc // 2
    rt = npad // NS
    ro = (rw // 2) // 8 * 8
    ic = ((rw - ro + 7) // 8) * 8

    mesh = plsc.VectorSubcoreMesh(
        core_axis_name="c", subcore_axis_name="s", num_cores=NC, num_subcores=NS
    )

    @functools.partial(
        pl.kernel,
        out_type=jax.ShapeDtypeStruct((NC, npad, D), jnp.float32),
        mesh=mesh,
        scratch_types=[
            pltpu.VMEM((2 * IW, D), jnp.float32),
            pltpu.VMEM((ic, IW), jnp.int32),
            pltpu.VMEM_SHARED((npad, D), jnp.float32),
            pltpu.SemaphoreType.DMA,
            pltpu.SemaphoreType.DMA,
        ],
    )
    def sc_agg(ea_hbm, di_hbm, out_hbm, rows_v, idx_v, acc_sh,
               sem_a, sem_b):
        c = lax.axis_index("c")
        s = lax.axis_index("s")
        wid = c * NS + s
        rows_a = rows_v.at[pl.ds(0, IW)]
        rows_b = rows_v.at[pl.ds(IW, IW)]

        pltpu.sync_copy(di_hbm.at[wid].at[pl.ds(0, ic)], idx_v)

        zero16 = jnp.zeros((16,), jnp.float32)

        def zbody(i, carry):
            for j in range(D // 16):
                rows_v[i, pl.ds(j * 16, 16)] = zero16
            return carry

        lax.fori_loop(0, ZR, zbody, 0)
        nfull = rt // ZR
        for r in range(nfull):
            pltpu.sync_copy(
                rows_v.at[pl.ds(0, ZR)],
                acc_sh.at[pl.ds(s * rt + r * ZR, ZR)],
            )
        if rt % ZR:
            pltpu.sync_copy(
                rows_v.at[pl.ds(0, rt % ZR)],
                acc_sh.at[pl.ds(s * rt + nfull * ZR, rt % ZR)],
            )
        plsc.subcore_barrier()

        def load(k, buf, sem):
            pltpu.async_copy(
                ea_hbm.at[pl.ds(wid * e_pt + k * IW, IW)], buf, sem
            )

        def wait_load(buf, sem):
            pltpu.make_async_copy(
                ea_hbm.at[pl.ds(0, IW)], buf, sem
            ).wait()

        def scatter(buf, k):
            irow = jnp.where(k < ro, k, k - ro)
            pltpu.sync_copy(
                buf,
                acc_sh.at[idx_v.at[irow]],
                add=True,
            )

        load(0, rows_a, sem_a)

        def pair(kk, carry):
            k0 = 2 * kk

            @pl.when(k0 == ro)
            def _():
                pltpu.sync_copy(
                    di_hbm.at[wid].at[pl.ds(ro, ic)],
                    idx_v,
                )

            load(k0 + 1, rows_b, sem_b)
            wait_load(rows_a, sem_a)
            scatter(rows_a, k0)

            @pl.when(k0 + 2 < nfc)
            def _():
                load(k0 + 2, rows_a, sem_a)

            wait_load(rows_b, sem_b)
            scatter(rows_b, k0 + 1)
            return carry

        lax.fori_loop(0, npair, pair, 0)
        if nfc % 2:
            wait_load(rows_a, sem_a)
            scatter(rows_a, nfc - 1)
        if te:
            tb = IW if nfc % 2 else 0
            pltpu.sync_copy(
                ea_hbm.at[pl.ds(wid * e_pt + nfc * IW, te)],
                rows_v.at[pl.ds(tb, te)],
            )

            def tzero(i, carry):
                for j in range(D // 16):
                    rows_v[tb + i, pl.ds(j * 16, 16)] = zero16
                return carry

            lax.fori_loop(te, IW, tzero, 0)
            scatter(rows_b if nfc % 2 else rows_a, rw - 1)
        plsc.subcore_barrier()

        pltpu.sync_copy(
            acc_sh.at[pl.ds(s * rt, rt)],
            out_hbm.at[c].at[pl.ds(s * rt, rt)],
        )

    return sc_agg(edge_attr, dest3d)


def _u_body(x_ref, f_ref, wx_ref, wf_ref, b0_ref, u_ref):
    u = jnp.dot(x_ref[...], wx_ref[...], preferred_element_type=jnp.float32)
    u = u + jnp.dot(f_ref[...], wf_ref[...],
                    preferred_element_type=jnp.float32)
    u_ref[...] = u + b0_ref[...]


def _mlp_body(u_ref, p0_ref, p1_ref, wa_ref, w1_ref, b1_ref, w2_ref, b2_ref,
              o_ref):
    agg = p0_ref[0] + p1_ref[0]
    h = u_ref[...] + jnp.dot(agg, wa_ref[...],
                             preferred_element_type=jnp.float32)
    h = h * jax.nn.sigmoid(h)
    h = jnp.dot(h, w1_ref[...],
                preferred_element_type=jnp.float32) + b1_ref[...]
    h = h * jax.nn.sigmoid(h)
    o_ref[...] = jnp.dot(h, w2_ref[...],
                         preferred_element_type=jnp.float32) + b2_ref[...]


@jax.jit
def kernel(x, edge_index, edge_attr, f, W0, b0, W1, b1, W2, b2):
    N, D = x.shape
    E = edge_attr.shape[0]
    F = f.shape[1]
    H = W1.shape[0]

    algn = NS * 8
    npad = ((N + algn - 1) // algn) * algn

    e_pt = E // NW
    nfc = e_pt // IW
    te = e_pt - nfc * IW
    rw = nfc + (1 if te else 0)
    rw_pad = ((rw + 7) // 8) * 8
    dest2 = edge_index[1].reshape(NW, e_pt)
    if te:
        dest2 = jnp.pad(dest2, ((0, 0), (0, rw * IW - e_pt)))
    dest3d = dest2.reshape(NW, rw, IW)
    if rw_pad != rw:
        dest3d = jnp.pad(dest3d, ((0, 0), (0, rw_pad - rw), (0, 0)))
    partials = _sc_segment_sum(edge_attr, dest3d, npad)

    R = 2000
    nb = N // R
    Wx = W0[:D]
    Wa = W0[D:D + H]
    Wf = W0[D + H:]
    b0r = b0.reshape(1, H)
    b1r = b1.reshape(1, H)
    b2r = b2.reshape(1, H)

    const = lambda i: (0, 0)
    row = lambda i: (i, 0)
    u = pl.pallas_call(
        _u_body,
        grid=(nb,),
        in_specs=[
            pl.BlockSpec((R, D), row),
            pl.BlockSpec((R, F), row),
            pl.BlockSpec((D, H), const),
            pl.BlockSpec((F, H), const),
            pl.BlockSpec((1, H), const),
        ],
        out_specs=pl.BlockSpec((R, H), row),
        out_shape=jax.ShapeDtypeStruct((N, H), jnp.float32),
    )(x, f, Wx, Wf, b0r)

    out = pl.pallas_call(
        _mlp_body,
        grid=(nb,),
        in_specs=[
            pl.BlockSpec((R, H), row),
            pl.BlockSpec((1, R, D), lambda i: (0, i, 0)),
            pl.BlockSpec((1, R, D), lambda i: (1, i, 0)),
            pl.BlockSpec((H, H), const),
            pl.BlockSpec((H, H), const),
            pl.BlockSpec((1, H), const),
            pl.BlockSpec((H, H), const),
            pl.BlockSpec((1, H), const),
        ],
        out_specs=pl.BlockSpec((R, H), row),
        out_shape=jax.ShapeDtypeStruct((N, H), jnp.float32),
    )(u, partials, partials, Wa, W1, b1r, W2, b2r)
    return out


# speedup vs baseline: 1.3153x; 1.0136x over previous
import functools

import jax
import jax.numpy as jnp
from jax import lax
from jax.experimental import pallas as pl
from jax.experimental.pallas import tpu as pltpu
from jax.experimental.pallas import tpu_sc as plsc

NC = 2
NS = 16
NW = NC * NS

IW = 120
ZR = 160


def _sc_segment_sum(edge_attr, dest3d, npad):
    E, D = edge_attr.shape
    e_pt = E // NW
    nfc = e_pt // IW
    te = e_pt - nfc * IW
    rw = nfc + (1 if te else 0)
    npair = nfc // 2
    rt = npad // NS
    ro = (rw // 2) // 8 * 8
    ic = ((rw - ro + 7) // 8) * 8

    mesh = plsc.VectorSubcoreMesh(
        core_axis_name="c", subcore_axis_name="s", num_cores=NC, num_subcores=NS
    )

    @functools.partial(
        pl.kernel,
        out_type=jax.ShapeDtypeStruct((NC, npad, D), jnp.float32),
        mesh=mesh,
        scratch_types=[
            pltpu.VMEM((2 * IW, D), jnp.float32),
            pltpu.VMEM((ic, IW), jnp.int32),
            pltpu.VMEM_SHARED((npad, D), jnp.float32),
            pltpu.SemaphoreType.DMA,
            pltpu.SemaphoreType.DMA,
        ],
    )
    def sc_agg(ea_hbm, di_hbm, out_hbm, rows_v, idx_v, acc_sh,
               sem_a, sem_b):
        c = lax.axis_index("c")
        s = lax.axis_index("s")
        wid = c * NS + s
        rows_a = rows_v.at[pl.ds(0, IW)]
        rows_b = rows_v.at[pl.ds(IW, IW)]

        pltpu.sync_copy(di_hbm.at[wid].at[pl.ds(0, ic)], idx_v)

        zero16 = jnp.zeros((16,), jnp.float32)

        def zbody(i, carry):
            for j in range(D // 16):
                rows_v[i, pl.ds(j * 16, 16)] = zero16
            return carry

        lax.fori_loop(0, ZR, zbody, 0)
        nfull = rt // ZR
        for r in range(nfull):
            pltpu.sync_copy(
                rows_v.at[pl.ds(0, ZR)],
                acc_sh.at[pl.ds(s * rt + r * ZR, ZR)],
            )
        if rt % ZR:
            pltpu.sync_copy(
                rows_v.at[pl.ds(0, rt % ZR)],
                acc_sh.at[pl.ds(s * rt + nfull * ZR, rt % ZR)],
            )
        plsc.subcore_barrier()

        def load(k, buf, sem):
            pltpu.async_copy(
                ea_hbm.at[pl.ds(wid * e_pt + k * IW, IW)], buf, sem
            )

        def wait_load(buf, sem):
            pltpu.make_async_copy(
                ea_hbm.at[pl.ds(0, IW)], buf, sem
            ).wait()

        def scatter(buf, k):
            irow = jnp.where(k < ro, k, k - ro)
            pltpu.sync_copy(
                buf,
                acc_sh.at[idx_v.at[irow]],
                add=True,
            )

        load(0, rows_a, sem_a)

        def pair(kk, carry):
            k0 = 2 * kk

            @pl.when(k0 == ro)
            def _():
                pltpu.sync_copy(
                    di_hbm.at[wid].at[pl.ds(ro, ic)],
                    idx_v,
                )

            load(k0 + 1, rows_b, sem_b)
            wait_load(rows_a, sem_a)
            scatter(rows_a, k0)

            @pl.when(k0 + 2 < nfc)
            def _():
                load(k0 + 2, rows_a, sem_a)

            wait_load(rows_b, sem_b)
            scatter(rows_b, k0 + 1)
            return carry

        lax.fori_loop(0, npair, pair, 0)
        if nfc % 2:
            wait_load(rows_a, sem_a)
            scatter(rows_a, nfc - 1)
        if te:
            tb = IW if nfc % 2 else 0
            pltpu.sync_copy(
                ea_hbm.at[pl.ds(wid * e_pt + nfc * IW, te)],
                rows_v.at[pl.ds(tb, te)],
            )

            def tzero(i, carry):
                for j in range(D // 16):
                    rows_v[tb + i, pl.ds(j * 16, 16)] = zero16
                return carry

            lax.fori_loop(te, IW, tzero, 0)
            scatter(rows_b if nfc % 2 else rows_a, rw - 1)
        plsc.subcore_barrier()

        pltpu.sync_copy(
            acc_sh.at[pl.ds(s * rt, rt)],
            out_hbm.at[c].at[pl.ds(s * rt, rt)],
        )

    return sc_agg(edge_attr, dest3d)


def _u_body(x_ref, f_ref, wx_ref, wf_ref, b0_ref, u_ref):
    u = jnp.dot(x_ref[...], wx_ref[...], preferred_element_type=jnp.float32)
    u = u + jnp.dot(f_ref[...], wf_ref[...],
                    preferred_element_type=jnp.float32)
    u_ref[...] = u + b0_ref[...]


def _mlp_body(u_ref, p0_ref, p1_ref, wa_ref, w1_ref, b1_ref, w2_ref, b2_ref,
              o_ref):
    agg = p0_ref[0] + p1_ref[0]
    h = u_ref[...] + jnp.dot(agg, wa_ref[...],
                             preferred_element_type=jnp.float32)
    h = h * jax.nn.sigmoid(h)
    h = jnp.dot(h, w1_ref[...],
                preferred_element_type=jnp.float32) + b1_ref[...]
    h = h * jax.nn.sigmoid(h)
    o_ref[...] = jnp.dot(h, w2_ref[...],
                         preferred_element_type=jnp.float32) + b2_ref[...]


@jax.jit
def kernel(x, edge_index, edge_attr, f, W0, b0, W1, b1, W2, b2):
    N, D = x.shape
    E = edge_attr.shape[0]
    F = f.shape[1]
    H = W1.shape[0]

    algn = NS * 8
    npad = ((N + algn - 1) // algn) * algn

    e_pt = E // NW
    nfc = e_pt // IW
    te = e_pt - nfc * IW
    rw = nfc + (1 if te else 0)
    rw_pad = ((rw + 7) // 8) * 8
    dest2 = edge_index[1].reshape(NW, e_pt)
    if te:
        dest2 = jnp.pad(dest2, ((0, 0), (0, rw * IW - e_pt)))
    dest3d = dest2.reshape(NW, rw, IW)
    if rw_pad != rw:
        dest3d = jnp.pad(dest3d, ((0, 0), (0, rw_pad - rw), (0, 0)))
    partials = _sc_segment_sum(edge_attr, dest3d, npad)

    R = 2000
    nb = N // R
    Wx = W0[:D]
    Wa = W0[D:D + H]
    Wf = W0[D + H:]
    b0r = b0.reshape(1, H)
    b1r = b1.reshape(1, H)
    b2r = b2.reshape(1, H)

    const = lambda i: (0, 0)
    row = lambda i: (i, 0)
    u = pl.pallas_call(
        _u_body,
        grid=(nb,),
        in_specs=[
            pl.BlockSpec((R, D), row),
            pl.BlockSpec((R, F), row),
            pl.BlockSpec((D, H), const),
            pl.BlockSpec((F, H), const),
            pl.BlockSpec((1, H), const),
        ],
        out_specs=pl.BlockSpec((R, H), row),
        out_shape=jax.ShapeDtypeStruct((N, H), jnp.float32),
    )(x, f, Wx, Wf, b0r)

    out = pl.pallas_call(
        _mlp_body,
        grid=(nb,),
        in_specs=[
            pl.BlockSpec((R, H), row),
            pl.BlockSpec((1, R, D), lambda i: (0, i, 0)),
            pl.BlockSpec((1, R, D), lambda i: (1, i, 0)),
            pl.BlockSpec((H, H), const),
            pl.BlockSpec((H, H), const),
            pl.BlockSpec((1, H), const),
            pl.BlockSpec((H, H), const),
            pl.BlockSpec((1, H), const),
        ],
        out_specs=pl.BlockSpec((R, H), row),
        out_shape=jax.ShapeDtypeStruct((N, H), jnp.float32),
    )(u, partials, partials, Wa, W1, b1r, W2, b2r)
    return out


# speedup vs baseline: 1.3225x; 1.0055x over previous
import functools

import jax
import jax.numpy as jnp
from jax import lax
from jax.experimental import pallas as pl
from jax.experimental.pallas import tpu as pltpu
from jax.experimental.pallas import tpu_sc as plsc

NC = 2
NS = 16
NW = NC * NS

IW = 128
ZR = 160


def _sc_segment_sum(edge_attr, dest3d, npad):
    E, D = edge_attr.shape
    e_pt = E // NW
    nfc = e_pt // IW
    te = e_pt - nfc * IW
    rw = nfc + (1 if te else 0)
    npair = nfc // 2
    rt = npad // NS
    ro = (rw // 2) // 8 * 8
    ic = ((rw - ro + 7) // 8) * 8

    mesh = plsc.VectorSubcoreMesh(
        core_axis_name="c", subcore_axis_name="s", num_cores=NC, num_subcores=NS
    )

    @functools.partial(
        pl.kernel,
        out_type=jax.ShapeDtypeStruct((NC, npad, D), jnp.float32),
        mesh=mesh,
        scratch_types=[
            pltpu.VMEM((2 * IW, D), jnp.float32),
            pltpu.VMEM((ic, IW), jnp.int32),
            pltpu.VMEM_SHARED((npad, D), jnp.float32),
            pltpu.SemaphoreType.DMA,
            pltpu.SemaphoreType.DMA,
        ],
    )
    def sc_agg(ea_hbm, di_hbm, out_hbm, rows_v, idx_v, acc_sh,
               sem_a, sem_b):
        c = lax.axis_index("c")
        s = lax.axis_index("s")
        wid = c * NS + s
        rows_a = rows_v.at[pl.ds(0, IW)]
        rows_b = rows_v.at[pl.ds(IW, IW)]

        pltpu.sync_copy(di_hbm.at[wid].at[pl.ds(0, ic)], idx_v)

        zero16 = jnp.zeros((16,), jnp.float32)

        def zbody(i, carry):
            for j in range(D // 16):
                rows_v[i, pl.ds(j * 16, 16)] = zero16
            return carry

        lax.fori_loop(0, ZR, zbody, 0)
        nfull = rt // ZR
        for r in range(nfull):
            pltpu.sync_copy(
                rows_v.at[pl.ds(0, ZR)],
                acc_sh.at[pl.ds(s * rt + r * ZR, ZR)],
            )
        if rt % ZR:
            pltpu.sync_copy(
                rows_v.at[pl.ds(0, rt % ZR)],
                acc_sh.at[pl.ds(s * rt + nfull * ZR, rt % ZR)],
            )
        plsc.subcore_barrier()

        def load(k, buf, sem):
            pltpu.async_copy(
                ea_hbm.at[pl.ds(wid * e_pt + k * IW, IW)], buf, sem
            )

        def wait_load(buf, sem):
            pltpu.make_async_copy(
                ea_hbm.at[pl.ds(0, IW)], buf, sem
            ).wait()

        def scatter(buf, k):
            irow = jnp.where(k < ro, k, k - ro)
            pltpu.sync_copy(
                buf,
                acc_sh.at[idx_v.at[irow]],
                add=True,
            )

        load(0, rows_a, sem_a)

        def pair(kk, carry):
            k0 = 2 * kk

            @pl.when(k0 == ro)
            def _():
                pltpu.sync_copy(
                    di_hbm.at[wid].at[pl.ds(ro, ic)],
                    idx_v,
                )

            load(k0 + 1, rows_b, sem_b)
            wait_load(rows_a, sem_a)
            scatter(rows_a, k0)

            @pl.when(k0 + 2 < nfc)
            def _():
                load(k0 + 2, rows_a, sem_a)

            wait_load(rows_b, sem_b)
            scatter(rows_b, k0 + 1)
            return carry

        lax.fori_loop(0, npair, pair, 0)
        if nfc % 2:
            wait_load(rows_a, sem_a)
            scatter(rows_a, nfc - 1)
        if te:
            tb = IW if nfc % 2 else 0
            pltpu.sync_copy(
                ea_hbm.at[pl.ds(wid * e_pt + nfc * IW, te)],
                rows_v.at[pl.ds(tb, te)],
            )

            def tzero(i, carry):
                for j in range(D // 16):
                    rows_v[tb + i, pl.ds(j * 16, 16)] = zero16
                return carry

            lax.fori_loop(te, IW, tzero, 0)
            scatter(rows_b if nfc % 2 else rows_a, rw - 1)
        plsc.subcore_barrier()

        pltpu.sync_copy(
            acc_sh.at[pl.ds(s * rt, rt)],
            out_hbm.at[c].at[pl.ds(s * rt, rt)],
        )

    return sc_agg(edge_attr, dest3d)


def _u_body(x_ref, f_ref, wx_ref, wf_ref, b0_ref, u_ref):
    u = jnp.dot(x_ref[...], wx_ref[...], preferred_element_type=jnp.float32)
    u = u + jnp.dot(f_ref[...], wf_ref[...],
                    preferred_element_type=jnp.float32)
    u_ref[...] = u + b0_ref[...]


def _mlp_body(u_ref, p0_ref, p1_ref, wa_ref, w1_ref, b1_ref, w2_ref, b2_ref,
              o_ref):
    agg = p0_ref[0] + p1_ref[0]
    h = u_ref[...] + jnp.dot(agg, wa_ref[...],
                             preferred_element_type=jnp.float32)
    h = h * jax.nn.sigmoid(h)
    h = jnp.dot(h, w1_ref[...],
                preferred_element_type=jnp.float32) + b1_ref[...]
    h = h * jax.nn.sigmoid(h)
    o_ref[...] = jnp.dot(h, w2_ref[...],
                         preferred_element_type=jnp.float32) + b2_ref[...]


@jax.jit
def kernel(x, edge_index, edge_attr, f, W0, b0, W1, b1, W2, b2):
    N, D = x.shape
    E = edge_attr.shape[0]
    F = f.shape[1]
    H = W1.shape[0]

    algn = NS * 8
    npad = ((N + algn - 1) // algn) * algn

    e_pt = E // NW
    nfc = e_pt // IW
    te = e_pt - nfc * IW
    rw = nfc + (1 if te else 0)
    rw_pad = ((rw + 7) // 8) * 8
    dest2 = edge_index[1].reshape(NW, e_pt)
    if te:
        dest2 = jnp.pad(dest2, ((0, 0), (0, rw * IW - e_pt)))
    dest3d = dest2.reshape(NW, rw, IW)
    if rw_pad != rw:
        dest3d = jnp.pad(dest3d, ((0, 0), (0, rw_pad - rw), (0, 0)))
    partials = _sc_segment_sum(edge_attr, dest3d, npad)

    R = 2000
    nb = N // R
    Wx = W0[:D]
    Wa = W0[D:D + H]
    Wf = W0[D + H:]
    b0r = b0.reshape(1, H)
    b1r = b1.reshape(1, H)
    b2r = b2.reshape(1, H)

    const = lambda i: (0, 0)
    row = lambda i: (i, 0)
    u = pl.pallas_call(
        _u_body,
        grid=(nb,),
        in_specs=[
            pl.BlockSpec((R, D), row),
            pl.BlockSpec((R, F), row),
            pl.BlockSpec((D, H), const),
            pl.BlockSpec((F, H), const),
            pl.BlockSpec((1, H), const),
        ],
        out_specs=pl.BlockSpec((R, H), row),
        out_shape=jax.ShapeDtypeStruct((N, H), jnp.float32),
    )(x, f, Wx, Wf, b0r)

    out = pl.pallas_call(
        _mlp_body,
        grid=(nb,),
        in_specs=[
            pl.BlockSpec((R, H), row),
            pl.BlockSpec((1, R, D), lambda i: (0, i, 0)),
            pl.BlockSpec((1, R, D), lambda i: (1, i, 0)),
            pl.BlockSpec((H, H), const),
            pl.BlockSpec((H, H), const),
            pl.BlockSpec((1, H), const),
            pl.BlockSpec((H, H), const),
            pl.BlockSpec((1, H), const),
        ],
        out_specs=pl.BlockSpec((R, H), row),
        out_shape=jax.ShapeDtypeStruct((N, H), jnp.float32),
    )(u, partials, partials, Wa, W1, b1r, W2, b2r)
    return out
